# Initial kernel scaffold; baseline (speedup 1.0000x reference)
import jax
import jax.numpy as jnp
from jax.experimental import pallas as pl


def kernel(text_seq, cat_features, emb_table, cat_W, cat_b, bn_cat_g, bn_cat_b, W1, b1, g1, be1, W2, b2, g2, be2, W3, b3):
    raise NotImplementedError("write your pallas kernel here")


# speedup vs baseline: 3.9584x; 3.9584x over previous
import functools

import jax
import jax.numpy as jnp
from jax import lax
from jax.experimental import pallas as pl
from jax.experimental.pallas import tpu as pltpu
from jax.experimental.pallas import tpu_sc as plsc

B = 4096
L = 200
LP = 208
EMB = 64
NC = 2
NS = 16
NW = NC * NS
ROWS_PER_W = B // NW
HALF = LP // 2


def _pooled_sums(emb_table, idx_padded):
    mesh = plsc.VectorSubcoreMesh(core_axis_name="c", subcore_axis_name="s")

    @functools.partial(
        pl.kernel,
        out_type=jax.ShapeDtypeStruct((B, EMB), jnp.float32),
        mesh=mesh,
        compiler_params=pltpu.CompilerParams(use_tc_tiling_on_sc=False),
        scratch_types=[
            pltpu.VMEM((ROWS_PER_W, LP), jnp.int32),
            pltpu.VMEM((LP, EMB), jnp.float32),
            pltpu.VMEM((ROWS_PER_W, EMB), jnp.float32),
            pltpu.SemaphoreType.DMA,
        ],
    )
    def sc_kernel(table_hbm, idx_hbm, out_hbm, idx_v, rows_v, out_v, sem):
        wid = lax.axis_index("s") * NC + lax.axis_index("c")
        base = wid * ROWS_PER_W
        pltpu.sync_copy(idx_hbm.at[pl.ds(base, ROWS_PER_W)], idx_v)

        @pl.loop(0, ROWS_PER_W)
        def _(r):
            c0 = pltpu.async_copy(
                table_hbm.at[idx_v.at[r, pl.ds(0, HALF)]],
                rows_v.at[pl.ds(0, HALF)], sem)
            c1 = pltpu.async_copy(
                table_hbm.at[idx_v.at[r, pl.ds(HALF, HALF)]],
                rows_v.at[pl.ds(HALF, HALF)], sem)
            c0.wait()
            c1.wait()

            def body(i, acc):
                return tuple(
                    acc[c] + rows_v[i, pl.ds(c * 16, 16)] for c in range(4))

            z = jnp.zeros((16,), jnp.float32)
            a = lax.fori_loop(0, LP, body, (z, z, z, z))
            for c in range(4):
                out_v[r, pl.ds(c * 16, 16)] = a[c]

        pltpu.sync_copy(out_v, out_hbm.at[pl.ds(base, ROWS_PER_W)])

    return sc_kernel(emb_table, idx_padded)


def _bn(x, gamma, beta, eps=1e-5):
    mean = jnp.mean(x, axis=0)
    var = jnp.var(x, axis=0)
    return gamma * (x - mean) / jnp.sqrt(var + eps) + beta


def _dense_body(ts_ref, pooled_ref, cat_ref, catW_ref, catb_ref, bg_ref,
                bb_ref, W1a_ref, W1b_ref, b1_ref, g1_ref, be1_ref, W2_ref,
                b2_ref, g2_ref, be2_ref, W3_ref, b3_ref, out_ref):
    hp = lax.Precision.HIGHEST
    cnt = jnp.sum((ts_ref[...] != 0).astype(jnp.float32), axis=1,
                  keepdims=True)
    text_feat = pooled_ref[...] / (cnt + 1e-9)
    cat = jnp.dot(cat_ref[...], catW_ref[...], precision=hp) + catb_ref[...]
    cat = jax.nn.relu(_bn(cat, bg_ref[...], bb_ref[...]))
    h = (jnp.dot(text_feat, W1a_ref[...], precision=hp)
         + jnp.dot(cat, W1b_ref[...], precision=hp) + b1_ref[...])
    h = jax.nn.relu(_bn(h, g1_ref[...], be1_ref[...]))
    h = jnp.dot(h, W2_ref[...], precision=hp) + b2_ref[...]
    h = jax.nn.relu(_bn(h, g2_ref[...], be2_ref[...]))
    out_ref[...] = jnp.dot(h, W3_ref[...], precision=hp) + b3_ref[...]


def kernel(text_seq, cat_features, emb_table, cat_W, cat_b, bn_cat_g,
           bn_cat_b, W1, b1, g1, be1, W2, b2, g2, be2, W3, b3):
    idx = text_seq.astype(jnp.int32)
    idx_p = jnp.pad(idx, ((0, 0), (0, LP - L)))
    pooled = _pooled_sums(emb_table, idx_p)

    row = lambda v: v.reshape(1, -1)
    out = pl.pallas_call(
        _dense_body,
        out_shape=jax.ShapeDtypeStruct((B, 1), jnp.float32),
    )(idx, pooled, cat_features, cat_W, row(cat_b), row(bn_cat_g),
      row(bn_cat_b), W1[:EMB], W1[EMB:], row(b1), row(g1), row(be1), W2,
      row(b2), row(g2), row(be2), W3, row(b3))
    return out
